# baseline (speedup 1.0000x reference)
import jax
import jax.numpy as jnp
from jax.experimental import pallas as pl
from jax.experimental.pallas import tpu as pltpu

K = 16
_EPS = 1e-5


def _make_fused(bsz, c, o, hw):
    half = bsz // 2

    def fused(x1_ref, x2_ref, idx1_ref, idx2_ref, w_ref, wm_ref, g_ref,
              b_ref, out_ref,
              x2s, sums_s, cnt_s, sumsq_s, a_s, scale_s, sems):
        i = pl.program_id(0)

        def one_hot(idx_ref):
            idx = idx_ref[0, 0]
            return (idx[None, :] ==
                    jax.lax.broadcasted_iota(jnp.int32, (K, hw), 0))

        @pl.when(i < half)
        def _stats():
            w = w_ref[...].astype(jnp.bfloat16)
            for bb, xref, iref in ((i, x1_ref, idx1_ref),
                                   (i + half, x2_ref, idx2_ref)):
                x = xref[0].astype(jnp.bfloat16)
                x2 = jax.lax.dot_general(
                    w, x, (((0,), (0,)), ((), ())),
                    preferred_element_type=jnp.float32)
                x2s[pl.ds(bb, 1)] = x2[None]
                ohb = one_hot(iref)
                oh = ohb.astype(jnp.bfloat16)
                sums = jax.lax.dot_general(
                    oh, x2.astype(jnp.bfloat16), (((1,), (1,)), ((), ())),
                    preferred_element_type=jnp.float32)
                sums_s[pl.ds(bb, 1)] = sums[None]
                cnt_s[pl.ds(bb, 1)] = jnp.sum(
                    ohb.astype(jnp.float32), axis=1)[None]
                sumsq_s[pl.ds(bb, 1)] = jnp.sum(x2 * x2, axis=1)[None]

        @pl.when(i == half - 1)
        def _graph():
            sums = sums_s[...]
            cnt = cnt_s[...]
            sumsq = sumsq_s[...]
            wm = wm_ref[...]
            denom = cnt + (cnt == 0).astype(jnp.float32)
            means = sums / denom[:, :, None]
            m = jax.lax.dot_general(wm, wm, (((1,), (1,)), ((), ())),
                                    preferred_element_type=jnp.float32)
            d = means[:, None, :, :] - means[:, :, None, :]
            dr = d.reshape(bsz * K * K, o)
            dm = jax.lax.dot_general(dr, m, (((1,), (0,)), ((), ())),
                                     preferred_element_type=jnp.float32)
            q = jnp.sum(dm * dr, axis=1).reshape(bsz, K, K)
            ii = jax.lax.broadcasted_iota(jnp.int32, (K, K), 0)
            jj = jax.lax.broadcasted_iota(jnp.int32, (K, K), 1)
            offdiag = (ii != jj).astype(jnp.float32)
            adjn = jnp.exp(-q) * offdiag[None]
            adjm = jnp.stack([
                jax.lax.dot_general(adjn[b], means[b],
                                    (((1,), (0,)), ((), ())),
                                    preferred_element_type=jnp.float32)
                for b in range(bsz)])
            n = jnp.sum(cnt)
            tot = (jnp.sum(sums, axis=(0, 1))
                   + jnp.sum(cnt[:, :, None] * adjm, axis=(0, 1)))
            totsq = (jnp.sum(sumsq, axis=0)
                     + 2.0 * jnp.sum(adjm * sums, axis=(0, 1))
                     + jnp.sum(cnt[:, :, None] * adjm * adjm, axis=(0, 1)))
            mu = tot / n
            var = totsq / n - mu * mu
            scale = g_ref[0] * jax.lax.rsqrt(var + _EPS)
            shift = b_ref[0] - mu * scale
            a_s[...] = adjm * scale[None, None, :] + shift[None, None, :]
            scale_s[...] = scale[None, :]

        @pl.when(i >= half)
        def _apply():
            j = i - half
            s = scale_s[0]
            for bb, iref in ((j, idx1_ref), (j + half, idx2_ref)):
                x2 = x2s[bb]
                oh = one_hot(iref).astype(jnp.float32)
                a = a_s[bb]
                g = jax.lax.dot_general(a, oh, (((0,), (0,)), ((), ())),
                                        preferred_element_type=jnp.float32)
                res = s[:, None] * x2 + g
                x2s[pl.ds(bb, 1)] = res[None]
                pltpu.make_async_copy(x2s.at[bb], out_ref.at[bb],
                                      sems.at[bb]).start()

        @pl.when(i == bsz - 1)
        def _drain():
            for b in range(bsz):
                pltpu.make_async_copy(x2s.at[b], out_ref.at[b],
                                      sems.at[b]).wait()

    return fused


def kernel(input, index, weight, W, bn_gamma, bn_beta):
    bsz, c, h, wsp = input.shape
    o = weight.shape[1]
    hw = h * wsp
    f32 = jnp.float32
    half = bsz // 2
    assert bsz % 2 == 0, "batch must be even"

    ih, iw = index.shape[2], index.shape[3]
    if (ih, iw) != (h, wsp):
        rows = (jnp.arange(h) * ih) // h
        cols = (jnp.arange(wsp) * iw) // wsp
        index = index[:, :, rows[:, None], cols[None, :]]
    idx3 = (index.reshape(bsz, 1, hw) - 1).astype(jnp.int32)
    xr = input.reshape(bsz, c, hw)

    stat_ix = lambda i: (jnp.minimum(i, half - 1), 0, 0)
    both_ix = lambda i: (jnp.where(i < half, i, i - half), 0, 0)

    out = pl.pallas_call(
        _make_fused(bsz, c, o, hw),
        grid=(bsz,),
        in_specs=[
            pl.BlockSpec((1, c, hw), stat_ix),
            pl.BlockSpec((1, c, hw), stat_ix),
            pl.BlockSpec((1, 1, hw), both_ix),
            pl.BlockSpec((1, 1, hw), both_ix),
            pl.BlockSpec((c, o), lambda i: (0, 0)),
            pl.BlockSpec((o, o), lambda i: (0, 0)),
            pl.BlockSpec((1, o), lambda i: (0, 0)),
            pl.BlockSpec((1, o), lambda i: (0, 0)),
        ],
        out_specs=pl.BlockSpec(memory_space=pltpu.MemorySpace.HBM),
        out_shape=jax.ShapeDtypeStruct((bsz, o, hw), f32),
        scratch_shapes=[
            pltpu.VMEM((bsz, o, hw), f32),
            pltpu.VMEM((bsz, K, o), f32),
            pltpu.VMEM((bsz, K), f32),
            pltpu.VMEM((bsz, o), f32),
            pltpu.VMEM((bsz, K, o), f32),
            pltpu.VMEM((1, o), f32),
            pltpu.SemaphoreType.DMA((bsz,)),
        ],
    )(xr[:half], xr[half:], idx3[:half], idx3[half:],
      weight, W, bn_gamma.reshape(1, o), bn_beta.reshape(1, o))

    return out.reshape(bsz, o, h, wsp)


# speedup vs baseline: 1.0093x; 1.0093x over previous
import jax
import jax.numpy as jnp
from jax.experimental import pallas as pl
from jax.experimental.pallas import tpu as pltpu

K = 16
_EPS = 1e-5


def _make_fused(bsz, c, o, hw):
    half = bsz // 2

    def fused(x1_ref, x2_ref, idxf_ref, w_ref, wm_ref, g_ref, b_ref,
              out_ref, x2s, sums_s, cnt_s, sumsq_s, a_s, scale_s):
        i = pl.program_id(0)

        def one_hot_f(bb, dtype):
            idx = idxf_ref[bb, 0]
            return (idx[None, :] ==
                    jax.lax.broadcasted_iota(jnp.int32, (K, hw), 0)
                    ).astype(dtype)

        @pl.when(i < half)
        def _stats():
            w = w_ref[...].astype(jnp.bfloat16)
            for bb, xref in ((i, x1_ref), (i + half, x2_ref)):
                x = xref[0].astype(jnp.bfloat16)
                x2 = jax.lax.dot_general(
                    w, x, (((0,), (0,)), ((), ())),
                    preferred_element_type=jnp.float32)
                x2s[pl.ds(bb, 1)] = x2[None]
                oh = one_hot_f(bb, jnp.float32)
                sums = jax.lax.dot_general(
                    oh.astype(jnp.bfloat16), x2.astype(jnp.bfloat16),
                    (((1,), (1,)), ((), ())),
                    preferred_element_type=jnp.float32)
                sums_s[pl.ds(bb, 1)] = sums[None]
                cnt_s[pl.ds(bb, 1)] = jnp.sum(oh, axis=1)[None]
                sumsq_s[pl.ds(bb, 1)] = jnp.sum(x2 * x2, axis=1)[None]

        @pl.when(i == half - 1)
        def _graph():
            sums = sums_s[...]
            cnt = cnt_s[...]
            sumsq = sumsq_s[...]
            wm = wm_ref[...]
            denom = cnt + (cnt == 0).astype(jnp.float32)
            means = sums / denom[:, :, None]
            m = jax.lax.dot_general(wm, wm, (((1,), (1,)), ((), ())),
                                    preferred_element_type=jnp.float32)
            d = means[:, None, :, :] - means[:, :, None, :]
            dr = d.reshape(bsz * K * K, o)
            dm = jax.lax.dot_general(dr, m, (((1,), (0,)), ((), ())),
                                     preferred_element_type=jnp.float32)
            q = jnp.sum(dm * dr, axis=1).reshape(bsz, K, K)
            ii = jax.lax.broadcasted_iota(jnp.int32, (K, K), 0)
            jj = jax.lax.broadcasted_iota(jnp.int32, (K, K), 1)
            offdiag = (ii != jj).astype(jnp.float32)
            adjn = jnp.exp(-q) * offdiag[None]
            adjm = jnp.stack([
                jax.lax.dot_general(adjn[b], means[b],
                                    (((1,), (0,)), ((), ())),
                                    preferred_element_type=jnp.float32)
                for b in range(bsz)])
            n = jnp.sum(cnt)
            tot = (jnp.sum(sums, axis=(0, 1))
                   + jnp.sum(cnt[:, :, None] * adjm, axis=(0, 1)))
            totsq = (jnp.sum(sumsq, axis=0)
                     + 2.0 * jnp.sum(adjm * sums, axis=(0, 1))
                     + jnp.sum(cnt[:, :, None] * adjm * adjm, axis=(0, 1)))
            mu = tot / n
            var = totsq / n - mu * mu
            scale = g_ref[0] * jax.lax.rsqrt(var + _EPS)
            shift = b_ref[0] - mu * scale
            a_s[...] = adjm * scale[None, None, :] + shift[None, None, :]
            scale_s[...] = scale[None, :]

        @pl.when(i >= half)
        def _apply():
            b = i - half
            x2 = x2s[b]
            oh = one_hot_f(b, jnp.float32)
            a = a_s[b]
            g = jax.lax.dot_general(a, oh, (((0,), (0,)), ((), ())),
                                    preferred_element_type=jnp.float32)
            out_ref[0] = scale_s[0][:, None] * x2 + g

    return fused


def kernel(input, index, weight, W, bn_gamma, bn_beta):
    bsz, c, h, wsp = input.shape
    o = weight.shape[1]
    hw = h * wsp
    f32 = jnp.float32
    half = bsz // 2
    assert bsz % 2 == 0, "batch must be even"

    ih, iw = index.shape[2], index.shape[3]
    if (ih, iw) != (h, wsp):
        rows = (jnp.arange(h) * ih) // h
        cols = (jnp.arange(wsp) * iw) // wsp
        index = index[:, :, rows[:, None], cols[None, :]]
    idx3 = (index.reshape(bsz, 1, hw) - 1).astype(jnp.int32)
    xr = input.reshape(bsz, c, hw)

    stat_ix = lambda i: (jnp.minimum(i, half - 1), 0, 0)

    out = pl.pallas_call(
        _make_fused(bsz, c, o, hw),
        grid=(half + bsz,),
        in_specs=[
            pl.BlockSpec((1, c, hw), stat_ix),
            pl.BlockSpec((1, c, hw), stat_ix),
            pl.BlockSpec((bsz, 1, hw), lambda i: (0, 0, 0)),
            pl.BlockSpec((c, o), lambda i: (0, 0)),
            pl.BlockSpec((o, o), lambda i: (0, 0)),
            pl.BlockSpec((1, o), lambda i: (0, 0)),
            pl.BlockSpec((1, o), lambda i: (0, 0)),
        ],
        out_specs=pl.BlockSpec((1, o, hw),
                               lambda i: (jnp.maximum(i - half, 0), 0, 0)),
        out_shape=jax.ShapeDtypeStruct((bsz, o, hw), f32),
        scratch_shapes=[
            pltpu.VMEM((bsz, o, hw), f32),
            pltpu.VMEM((bsz, K, o), f32),
            pltpu.VMEM((bsz, K), f32),
            pltpu.VMEM((bsz, o), f32),
            pltpu.VMEM((bsz, K, o), f32),
            pltpu.VMEM((1, o), f32),
        ],
    )(xr[:half], xr[half:], idx3, weight, W,
      bn_gamma.reshape(1, o), bn_beta.reshape(1, o))

    return out.reshape(bsz, o, h, wsp)
